# baseline (speedup 1.0000x reference)
import jax
import jax.numpy as jnp
from jax.experimental import pallas as pl
from jax.experimental.pallas import tpu as pltpu

_T, _B, _D, _H, _MC, _ES = 32, 64, 256, 256, 1024, 256


def _masst_body(x_ref, gu_ref, whh_ref, bhh_ref, wcat_ref, wum_ref,
                fc1w_ref, fc1b_ref, fc2b_ref,
                wih_ref, bih_ref, wim_ref,
                out_ref, histw_ref):
    x_all = x_ref[...].reshape(_T * _B, _D)
    wi_all = x_all @ wih_ref[...] + bih_ref[...]
    xim_all = x_all @ wim_ref[...]

    whh = whh_ref[...]
    bhh = bhh_ref[...]
    wcat = wcat_ref[...]
    wum = wum_ref[...]
    fc1w = fc1w_ref[...]
    fc1b = fc1b_ref[...]
    fc2b = fc2b_ref[...]

    iota_mc = jax.lax.broadcasted_iota(jnp.int32, (_B, _MC), 1)
    iota_128 = jax.lax.broadcasted_iota(jnp.int32, (_B, 128), 1)

    sig_m1 = jax.nn.sigmoid(jnp.full((_B, _MC), -1.0, jnp.float32))

    def step(t, h, lu, last_use):
        hw = h @ wcat
        whm_p = hw[:, :_H]
        histw_ref[t] = hw[:, _H:2 * _H]
        wh_p = hw[:, 2 * _H:]
        pre = jnp.tanh(xim_all[t * _B:(t + 1) * _B] + whm_p + last_use @ wum)
        read_head = pre @ fc1w + fc1b
        g = -jnp.log(1e-20 - jnp.log(1e-20 + gu_ref[t]))
        logits = read_head + g
        bmax = []
        bpos = []
        for c in range(_MC // 128):
            blk = logits[:, c * 128:(c + 1) * 128]
            bm = jnp.max(blk, axis=1, keepdims=True)
            bp = jnp.min(jnp.where(blk == bm, iota_128, 128), axis=1,
                         keepdims=True) + (c * 128)
            bmax.append(bm)
            bpos.append(bp)
        m = bmax[0]
        for bm in bmax[1:]:
            m = jnp.maximum(m, bm)
        pos = jnp.full((_B, 1), _MC, jnp.int32)
        for bm, bp in zip(reversed(bmax), reversed(bpos)):
            pos = jnp.where(bm == m, bp, pos)
        hot = iota_mc == pos
        lu_dec = lu - 1.0
        sp = jax.nn.sigmoid(lu_dec)
        lu_next = jnp.where(hot, -1.0, lu_dec)
        last_use_next = jnp.where(hot, sig_m1, sp)
        posH = jnp.broadcast_to(pos, (_B, _H))
        terms = [jnp.where(posH == j, histw_ref[j], 0.0)
                 for j in range(t + 1)]
        while len(terms) > 1:
            terms = [a + b for a, b in zip(terms[::2], terms[1::2])] + (
                [terms[-1]] if len(terms) % 2 else [])
        gw = terms[0]
        h_new = gw + wh_p + fc2b
        wh_b = h_new @ whh + bhh
        wi_b = wi_all[t * _B:(t + 1) * _B]
        r = jax.nn.sigmoid(wi_b[:, :_H] + wh_b[:, :_H])
        z = jax.nn.sigmoid(wi_b[:, _H:2 * _H] + wh_b[:, _H:2 * _H])
        n = jnp.tanh(wi_b[:, 2 * _H:] + r * wh_b[:, 2 * _H:])
        h2 = (1.0 - z) * n + z * h_new
        out_ref[t] = h2
        return h2, lu_next, last_use_next

    h = jnp.zeros((_B, _H), jnp.float32)
    lu = jnp.full((_B, _MC), -99999.0, jnp.float32)
    last_use = jax.nn.sigmoid(lu)
    for t in range(_T):
        h, lu, last_use = step(t, h, lu, last_use)


def kernel(input_, gumbel_u, weight_ih, weight_hh, bias, weight_im,
           weight_hm, weight_um, fc1_w, fc1_b, fc2_w, fc2_b):
    bih = bias[: 3 * _H].reshape(1, 3 * _H)
    bhh = bias[3 * _H:].reshape(1, 3 * _H)
    fc1b = fc1_b.reshape(1, _MC)
    fc2b = fc2_b.reshape(1, _H)
    wcat = jnp.concatenate([weight_hm, fc2_w[:_ES], fc2_w[_ES:]], axis=1)
    return pl.pallas_call(
        _masst_body,
        out_shape=jax.ShapeDtypeStruct((_T, _B, _H), jnp.float32),
        scratch_shapes=[
            pltpu.VMEM((_T, _B, _H), jnp.float32),
        ],
        compiler_params=pltpu.CompilerParams(
            vmem_limit_bytes=100 * 1024 * 1024,
        ),
    )(input_, gumbel_u, weight_hh, bhh, wcat, weight_um,
      fc1_w, fc1b, fc2b, weight_ih, bih, weight_im)


# speedup vs baseline: 1.0787x; 1.0787x over previous
import jax
import jax.numpy as jnp
from jax.experimental import pallas as pl
from jax.experimental.pallas import tpu as pltpu

_T, _B, _D, _H, _MC, _ES = 32, 64, 256, 256, 1024, 256


def _masst_body(x_ref, gu_ref, whh_ref, bhh_ref, wcat_ref, wum_ref,
                fc1w_ref, fc1b_ref, fc2b_ref,
                wih_ref, bih_ref, wim_ref,
                out_ref, histw_ref):
    x_all = x_ref[...].reshape(_T * _B, _D)
    wi_all = x_all @ wih_ref[...] + bih_ref[...]
    xim_all = x_all @ wim_ref[...]

    whh = whh_ref[...]
    bhh = bhh_ref[...]
    wcat = wcat_ref[...]
    wum = wum_ref[...]
    fc1w = fc1w_ref[...]
    fc1b = fc1b_ref[...]
    fc2b = fc2b_ref[...]

    iota_mc = jax.lax.broadcasted_iota(jnp.int32, (_B, _MC), 1)

    sig_m1 = jax.nn.sigmoid(jnp.full((_B, _MC), -1.0, jnp.float32))

    def step(t, h, lu, luw):
        hw = h @ wcat
        whm_p = hw[:, :_H]
        histw_ref[t] = hw[:, _H:2 * _H]
        wh_p = hw[:, 2 * _H:]
        pre = jnp.tanh(xim_all[t * _B:(t + 1) * _B] + whm_p + luw)
        read_head = pre @ fc1w + fc1b
        g = -jnp.log(1e-20 - jnp.log(1e-20 + gu_ref[t]))
        logits = read_head + g
        m = jnp.max(logits, axis=1, keepdims=True)
        pos = jnp.min(jnp.where(logits == m, iota_mc, _MC), axis=1,
                      keepdims=True)
        hot = iota_mc == pos
        lu_dec = lu - 1.0
        sp = jax.nn.sigmoid(lu_dec)
        lu_next = jnp.where(hot, -1.0, lu_dec)
        luw_next = jnp.where(hot, sig_m1, sp) @ wum
        posH = jnp.broadcast_to(pos, (_B, _H))
        terms = [jnp.where(posH == j, histw_ref[j], 0.0)
                 for j in range(t + 1)]
        while len(terms) > 1:
            terms = [a + b for a, b in zip(terms[::2], terms[1::2])] + (
                [terms[-1]] if len(terms) % 2 else [])
        gw = terms[0]
        h_new = gw + wh_p + fc2b
        wh_b = h_new @ whh + bhh
        wi_b = wi_all[t * _B:(t + 1) * _B]
        r = jax.nn.sigmoid(wi_b[:, :_H] + wh_b[:, :_H])
        z = jax.nn.sigmoid(wi_b[:, _H:2 * _H] + wh_b[:, _H:2 * _H])
        n = jnp.tanh(wi_b[:, 2 * _H:] + r * wh_b[:, 2 * _H:])
        h2 = (1.0 - z) * n + z * h_new
        out_ref[t] = h2
        return h2, lu_next, luw_next

    h = jnp.zeros((_B, _H), jnp.float32)
    lu = jnp.full((_B, _MC), -99999.0, jnp.float32)
    luw = jax.nn.sigmoid(lu) @ wum
    for t in range(_T):
        h, lu, luw = step(t, h, lu, luw)


def kernel(input_, gumbel_u, weight_ih, weight_hh, bias, weight_im,
           weight_hm, weight_um, fc1_w, fc1_b, fc2_w, fc2_b):
    bih = bias[: 3 * _H].reshape(1, 3 * _H)
    bhh = bias[3 * _H:].reshape(1, 3 * _H)
    fc1b = fc1_b.reshape(1, _MC)
    fc2b = fc2_b.reshape(1, _H)
    wcat = jnp.concatenate([weight_hm, fc2_w[:_ES], fc2_w[_ES:]], axis=1)
    return pl.pallas_call(
        _masst_body,
        out_shape=jax.ShapeDtypeStruct((_T, _B, _H), jnp.float32),
        scratch_shapes=[
            pltpu.VMEM((_T, _B, _H), jnp.float32),
        ],
        compiler_params=pltpu.CompilerParams(
            vmem_limit_bytes=100 * 1024 * 1024,
        ),
    )(input_, gumbel_u, weight_hh, bhh, wcat, weight_um,
      fc1_w, fc1b, fc2b, weight_ih, bih, weight_im)
